# baseline (speedup 1.0000x reference)
import numpy as np
import jax
import jax.numpy as jnp
from jax.experimental import pallas as pl

_PE_L = 6
_H = 32
_B = 4096
_R = _B // 32


def _volume_body(x_ref, y_ref, scale_ref, off_ref, w1_ref, b1_ref, w2_ref,
                 wy_ref, selc_ref, seld_ref, bd_ref, bc_ref, d_ref, c_ref):
    X = x_ref[:]
    ndc = X * scale_ref[:] + off_ref[:]
    inb = ((ndc >= -1.0) & (ndc <= 1.0)).astype(jnp.bfloat16)
    cnt_c = jnp.dot(inb, selc_ref[:],
                    preferred_element_type=jnp.float32)
    cnt_d = jnp.dot(inb, seld_ref[:],
                    preferred_element_type=jnp.float32)

    T = ndc.T
    s = jnp.sin(jnp.pi * T)
    c = jnp.cos(jnp.pi * T)
    feats = [T, s, c]
    for _ in range(1, _PE_L):
        s, c = 2.0 * s * c, 1.0 - 2.0 * s * s
        feats.append(s)
        feats.append(c)

    pe_cols = []
    for g in range(8):
        pe_cols.append(jnp.concatenate(
            [f[12 * g:12 * (g + 1), :] for f in feats], axis=0))
    pe_wide = jnp.concatenate(pe_cols, axis=1).astype(jnp.bfloat16)
    f_wide = jnp.dot(w1_ref[:], pe_wide,
                     preferred_element_type=jnp.float32)
    f_wide = jnp.maximum(f_wide + b1_ref[:], 0.0)

    f2_parts = []
    for j in range(4):
        f2_parts.append(jnp.concatenate(
            [f_wide[:, (2 * j) * _R:(2 * j + 1) * _R],
             f_wide[:, (2 * j + 1) * _R:(2 * j + 2) * _R]],
            axis=0))
    f2_wide = jnp.concatenate(f2_parts, axis=1).astype(jnp.bfloat16)
    o_wide = jnp.dot(w2_ref[:], f2_wide,
                     preferred_element_type=jnp.float32)
    o_d = jnp.concatenate(
        [o_wide[0:8, j * _R:(j + 1) * _R] for j in range(4)], axis=0)
    o_c = jnp.concatenate(
        [o_wide[8:32, j * _R:(j + 1) * _R] for j in range(4)], axis=0)

    o_y = jnp.dot(y_ref[:].astype(jnp.bfloat16), wy_ref[:],
                  preferred_element_type=jnp.float32)

    od = o_d.T + bd_ref[:]
    d = jnp.maximum(od, 0.0) + jnp.log1p(jnp.exp(-jnp.abs(od)))
    d_ref[:] = jnp.where(cnt_d > 2.5, d, 0.0)

    oc = o_c.T + o_y + bc_ref[:]
    col = 1.0 / (1.0 + jnp.exp(-oc))
    c_ref[:] = jnp.where(cnt_c > 2.5, col, 0.0)


def kernel(xyz, ynm, aabb, W1, b1, Wd, bd, Wc, bc):
    N, S, _ = xyz.shape
    P = N * S
    R = _R
    P32 = P // 32

    X32 = xyz.reshape(P32, 96)
    Y32 = ynm.reshape(P32, 512)

    rng = aabb[1] - aabb[0]
    scale96 = jnp.tile(2.0 / rng, 32).reshape(1, 96)
    off96 = jnp.tile(-2.0 * aabb[0] / rng - 1.0, 32).reshape(1, 96)

    perm = np.empty(39, np.int64)
    for F in range(13):
        for k in range(3):
            if F == 0:
                j = k
            elif F % 2 == 1:
                j = 3 + 6 * ((F - 1) // 2) + k
            else:
                j = 6 + 6 * ((F - 2) // 2) + k
            perm[3 * F + k] = j
    WpT = W1.T[:, perm]
    w1p = jnp.zeros((128, 156), jnp.float32)
    for q in range(4):
        cols = np.array([12 * F + 3 * q + k
                         for F in range(13) for k in range(3)])
        w1p = w1p.at[32 * q:32 * (q + 1), cols].set(WpT)
    w1p = w1p.astype(jnp.bfloat16)
    b1col = jnp.tile(b1, 4).reshape(128, 1)

    w2f = jnp.zeros((32, 256), jnp.float32)
    for jj in range(2):
        for q in range(4):
            p_loc = 4 * jj + q
            cb = 128 * jj + 32 * q
            w2f = w2f.at[p_loc, cb:cb + 32].set(Wd[:, 0])
            w2f = w2f.at[8 + 3 * p_loc:8 + 3 * p_loc + 3, cb:cb + 32].set(
                Wc[:32].T)
    w2f = w2f.astype(jnp.bfloat16)

    wy = jnp.zeros((512, 96), jnp.float32)
    for q in range(32):
        wy = wy.at[16 * q:16 * q + 16, 3 * q:3 * q + 3].set(Wc[32:])
    wy = wy.astype(jnp.bfloat16)

    selc = jnp.asarray(np.kron(np.eye(32), np.ones((3, 3))), jnp.bfloat16)
    seld = jnp.asarray(np.kron(np.eye(32), np.ones((3, 1))), jnp.bfloat16)
    bdc = bd.reshape(1, 1)
    bc96 = jnp.tile(bc, 32).reshape(1, 96)

    grid = P32 // R
    dO, cO = pl.pallas_call(
        _volume_body,
        grid=(grid,),
        in_specs=[
            pl.BlockSpec((R, 96), lambda i: (i, 0)),
            pl.BlockSpec((R, 512), lambda i: (i, 0)),
            pl.BlockSpec((1, 96), lambda i: (0, 0)),
            pl.BlockSpec((1, 96), lambda i: (0, 0)),
            pl.BlockSpec((128, 156), lambda i: (0, 0)),
            pl.BlockSpec((128, 1), lambda i: (0, 0)),
            pl.BlockSpec((32, 256), lambda i: (0, 0)),
            pl.BlockSpec((512, 96), lambda i: (0, 0)),
            pl.BlockSpec((96, 96), lambda i: (0, 0)),
            pl.BlockSpec((96, 32), lambda i: (0, 0)),
            pl.BlockSpec((1, 1), lambda i: (0, 0)),
            pl.BlockSpec((1, 96), lambda i: (0, 0)),
        ],
        out_specs=[
            pl.BlockSpec((R, 32), lambda i: (i, 0)),
            pl.BlockSpec((R, 96), lambda i: (i, 0)),
        ],
        out_shape=[
            jax.ShapeDtypeStruct((P32, 32), jnp.float32),
            jax.ShapeDtypeStruct((P32, 96), jnp.float32),
        ],
    )(X32, Y32, scale96, off96, w1p, b1col, w2f, wy, selc, seld, bdc, bc96)

    density = dO.reshape(N, S, 1)
    color = cO.reshape(N, S, 3)
    return density, color


# speedup vs baseline: 7.0613x; 7.0613x over previous
import jax
import jax.numpy as jnp
from jax.experimental import pallas as pl

_PE_L = 6
_H = 32
_B = 4096
_G = 4

_S1 = 3.141592653589793
_S3 = -5.167712780049970
_S5 = 2.550164039877345
_S7 = -0.5992645293207921
_S9 = 0.0821458866111282
_S11 = -0.0073704309457144
_C0 = 1.0
_C2 = -4.934802200544679
_C4 = 4.058712126416768
_C6 = -1.3352627688545895
_C8 = 0.2353306303588932
_C10 = -0.0258068327360992


def _sincos_pi(t):
    n = jnp.floor(t + 0.5)
    r = t - n
    z = r * r
    ps = ((((_S11 * z + _S9) * z + _S7) * z + _S5) * z + _S3) * z + _S1
    ps = ps * r
    pc = ((((_C10 * z + _C8) * z + _C6) * z + _C4) * z + _C2) * z + _C0
    an = jnp.abs(n)
    sgn = 1.0 - 2.0 * (an - 2.0 * jnp.floor(an * 0.5))
    return ps * sgn, pc * sgn


def _volume_body(xyzT_ref, ynmT_ref, scale_ref, off_ref, w1_ref, b1_ref,
                 w2_ref, b2_ref, dT_ref, cT_ref):
    B = _B
    C = B // _G
    t = xyzT_ref[:] * scale_ref[:] + off_ref[:]
    inb = (t >= -1.0) & (t <= 1.0)
    mask = inb[0:1] & inb[1:2] & inb[2:3]

    s, c = _sincos_pi(t)
    feats = [t, s, c]
    for _ in range(1, _PE_L):
        s, c = 2.0 * s * c, 1.0 - 2.0 * s * s
        feats.append(s)
        feats.append(c)
    pe = jnp.concatenate(feats, axis=0)

    pe_g = jnp.concatenate([pe[:, i * C:(i + 1) * C] for i in range(_G)],
                           axis=0).astype(jnp.bfloat16)
    f_g = jnp.dot(w1_ref[:], pe_g,
                  preferred_element_type=jnp.float32)
    f_g = jnp.maximum(f_g + b1_ref[:], 0.0)

    ynmT = ynmT_ref[:]
    z_parts = []
    for i in range(_G):
        z_parts.append(f_g[_H * i:_H * (i + 1), :].astype(jnp.bfloat16))
        z_parts.append(ynmT[:, i * C:(i + 1) * C])
    z_g = jnp.concatenate(z_parts, axis=0)
    o_g = jnp.dot(w2_ref[:], z_g,
                  preferred_element_type=jnp.float32) + b2_ref[:]
    o = jnp.concatenate([o_g[4 * i:4 * (i + 1), :] for i in range(_G)],
                        axis=1)

    row = jax.lax.broadcasted_iota(jnp.int32, (4, B), 0)
    sig = 1.0 / (1.0 + jnp.exp(-o))
    sp = jnp.maximum(o, 0.0) + jnp.log1p(jnp.exp(-jnp.abs(o)))
    act = jnp.where(row == 0, sp, sig)
    act = jnp.where(mask, act, 0.0)
    dT_ref[:] = act[0:1, :]
    cT_ref[:] = act[1:4, :]


def kernel(xyz, ynm, aabb, W1, b1, Wd, bd, Wc, bc):
    N, S, _ = xyz.shape
    P = N * S
    B, G = _B, _G

    xyzT = xyz.reshape(P, 3).T
    ynmT = ynm.reshape(P, 16).T.astype(jnp.bfloat16)
    rng = aabb[1] - aabb[0]
    scale = (2.0 / rng).reshape(3, 1)
    off = (-2.0 * aabb[0] / rng - 1.0).reshape(3, 1)

    W1T = W1.T
    w1bd = jax.scipy.linalg.block_diag(*([W1T] * G)).astype(jnp.bfloat16)
    b1t = jnp.tile(b1, G).reshape(G * _H, 1)
    W2 = jnp.concatenate(
        [jnp.concatenate([Wd, jnp.zeros((16, 1), jnp.float32)], axis=0), Wc],
        axis=1)
    w2bd = jax.scipy.linalg.block_diag(*([W2.T] * G)).astype(jnp.bfloat16)
    b2t = jnp.tile(jnp.concatenate([bd, bc]), G).reshape(4 * G, 1)

    grid = P // B
    dT, cT = pl.pallas_call(
        _volume_body,
        grid=(grid,),
        in_specs=[
            pl.BlockSpec((3, B), lambda i: (0, i)),
            pl.BlockSpec((16, B), lambda i: (0, i)),
            pl.BlockSpec((3, 1), lambda i: (0, 0)),
            pl.BlockSpec((3, 1), lambda i: (0, 0)),
            pl.BlockSpec(w1bd.shape, lambda i: (0, 0)),
            pl.BlockSpec((G * _H, 1), lambda i: (0, 0)),
            pl.BlockSpec(w2bd.shape, lambda i: (0, 0)),
            pl.BlockSpec((4 * G, 1), lambda i: (0, 0)),
        ],
        out_specs=[
            pl.BlockSpec((1, B), lambda i: (0, i)),
            pl.BlockSpec((3, B), lambda i: (0, i)),
        ],
        out_shape=[
            jax.ShapeDtypeStruct((1, P), jnp.float32),
            jax.ShapeDtypeStruct((3, P), jnp.float32),
        ],
    )(xyzT, ynmT, scale, off, w1bd, b1t, w2bd, b2t)

    density = dT.reshape(N, S, 1)
    color = cT.T.reshape(N, S, 3)
    return density, color
